# Initial kernel scaffold; baseline (speedup 1.0000x reference)
import jax
import jax.numpy as jnp
from jax.experimental import pallas as pl


def kernel(input_ids, table):
    raise NotImplementedError("write your pallas kernel here")


# speedup vs baseline: 5.0366x; 5.0366x over previous
import jax
import jax.numpy as jnp
from jax.experimental import pallas as pl


def _broadcast_body(table_ref, out_ref):
    out_ref[...] = jnp.broadcast_to(table_ref[...][None], out_ref.shape)


def kernel(input_ids, table):
    batch, seq_len = input_ids.shape
    max_pos, d_model = table.shape
    bs = 512
    out = pl.pallas_call(
        _broadcast_body,
        grid=(seq_len // bs,),
        in_specs=[pl.BlockSpec((bs, d_model), lambda i: (i, 0))],
        out_specs=pl.BlockSpec((batch, bs, d_model), lambda i: (0, i, 0)),
        out_shape=jax.ShapeDtypeStruct((batch, seq_len, d_model), table.dtype),
    )(table)
    return out
